# baseline (speedup 1.0000x reference)
import functools
import math

import jax
import jax.numpy as jnp
from jax import lax
from jax.experimental import pallas as pl
from jax.experimental.pallas import tpu as pltpu
from jax.experimental.pallas import tpu_sc as plsc

_IDX_MINOR = 128
_CHUNK = 128
_NBUF = 3


@functools.cache
def _make(B, V, DP, D):
    info = plsc.get_sparse_core_info()
    nw = info.num_cores * info.num_subcores
    b_per_w = B // nw
    idx_rows_per_w = b_per_w // _IDX_MINOR
    n_chunks = b_per_w // _CHUNK
    assert _CHUNK * nw * n_chunks == B
    n_outer = (n_chunks + _NBUF - 1) // _NBUF

    mesh = plsc.VectorSubcoreMesh(core_axis_name="c", subcore_axis_name="s")
    scale = float(math.sqrt(D))

    @functools.partial(
        pl.kernel,
        out_type=jax.ShapeDtypeStruct((B, D), jnp.float32),
        mesh=mesh,
        scratch_types=[
            pltpu.VMEM((idx_rows_per_w, _IDX_MINOR), jnp.int32),
            pltpu.VMEM((_NBUF, _CHUNK, DP), jnp.float32),
            pltpu.VMEM((_NBUF, _CHUNK, D), jnp.float32),
            pltpu.SemaphoreType.DMA((_NBUF,)),
            pltpu.SemaphoreType.DMA((_NBUF,)),
        ],
        compiler_params=pltpu.CompilerParams(use_tc_tiling_on_sc=True),
    )
    def emb_kernel(x_hbm, w_hbm, out_hbm, idx_v, rows_v, comp_v, gsem, osem):
        wid = lax.axis_index("s") * info.num_cores + lax.axis_index("c")
        idx_row_base = wid * idx_rows_per_w
        out_base = wid * b_per_w

        pltpu.sync_copy(x_hbm.at[pl.ds(idx_row_base, idx_rows_per_w)], idx_v)

        def issue_gather(g, b):
            pltpu.async_copy(w_hbm.at[idx_v.at[g]], rows_v.at[b], gsem.at[b])

        def wait_gather(b):
            pltpu.make_async_copy(
                w_hbm.at[pl.ds(0, _CHUNK)], rows_v.at[b], gsem.at[b]
            ).wait()

        def issue_out(g, b):
            pltpu.async_copy(
                comp_v.at[b], out_hbm.at[pl.ds(out_base + g * _CHUNK, _CHUNK)],
                osem.at[b],
            )

        def wait_out(b):
            pltpu.make_async_copy(
                comp_v.at[b], out_hbm.at[pl.ds(0, _CHUNK)], osem.at[b]
            ).wait()

        issue_gather(jnp.int32(0), 0)

        def outer(p, carry):
            for b in range(_NBUF):
                g = p * _NBUF + b

                @pl.when(g < n_chunks)
                def _():
                    wait_gather(b)

                    g2 = g + 1
                    b2 = (b + 1) % _NBUF

                    @pl.when(g2 < n_chunks)
                    def _():
                        @pl.when(g2 >= _NBUF)
                        def _():
                            wait_out(b2)

                        issue_gather(g2, b2)

                    @plsc.parallel_loop(0, _CHUNK, step=1, unroll=8)
                    def _(r):
                        for c in range(D // 16):
                            s = pl.ds(c * 16, 16)
                            comp_v[b, r, s] = rows_v[b, r, s] * scale

                    issue_out(g, b)
            return carry

        lax.fori_loop(0, n_outer, outer, 0)

        for b in range(_NBUF):
            wait_out(b)

    return emb_kernel


def kernel(x, W):
    b, h = x.shape
    V, D = W.shape
    flat = x.reshape(b * h // _IDX_MINOR, _IDX_MINOR).astype(jnp.int32)
    w_pad = jnp.concatenate(
        [W, jnp.zeros((V, 128 - D), dtype=W.dtype)], axis=1
    )
    out = _make(b * h, V, 128, D)(flat, w_pad)
    return out.reshape(b, h, D)


# speedup vs baseline: 1.0533x; 1.0533x over previous
import functools
import math

import jax
import jax.numpy as jnp
from jax import lax
from jax.experimental import pallas as pl
from jax.experimental.pallas import tpu as pltpu
from jax.experimental.pallas import tpu_sc as plsc

_IDX_MINOR = 128
_CHUNK = 128
_NBUF = 3
_AHEAD = 2


@functools.cache
def _make(B, V, DP, D):
    info = plsc.get_sparse_core_info()
    nw = info.num_cores * info.num_subcores
    b_per_w = B // nw
    idx_rows_per_w = b_per_w // _IDX_MINOR
    n_chunks = b_per_w // _CHUNK
    assert _CHUNK * nw * n_chunks == B
    n_outer = (n_chunks + _NBUF - 1) // _NBUF

    mesh = plsc.VectorSubcoreMesh(core_axis_name="c", subcore_axis_name="s")
    scale = float(math.sqrt(D))

    @functools.partial(
        pl.kernel,
        out_type=jax.ShapeDtypeStruct((B, D), jnp.float32),
        mesh=mesh,
        scratch_types=[
            pltpu.VMEM((idx_rows_per_w, _IDX_MINOR), jnp.int32),
            pltpu.VMEM((_NBUF, _CHUNK, DP), jnp.float32),
            pltpu.VMEM((_NBUF, _CHUNK, D), jnp.float32),
            pltpu.SemaphoreType.DMA((_NBUF,)),
            pltpu.SemaphoreType.DMA((_NBUF,)),
        ],
        compiler_params=pltpu.CompilerParams(use_tc_tiling_on_sc=True),
    )
    def emb_kernel(x_hbm, w_hbm, out_hbm, idx_v, rows_v, comp_v, gsem, osem):
        wid = lax.axis_index("s") * info.num_cores + lax.axis_index("c")
        idx_row_base = wid * idx_rows_per_w
        out_base = wid * b_per_w

        pltpu.sync_copy(x_hbm.at[pl.ds(idx_row_base, idx_rows_per_w)], idx_v)

        def issue_gather(g, b):
            pltpu.async_copy(w_hbm.at[idx_v.at[g]], rows_v.at[b], gsem.at[b])

        def wait_gather(b):
            pltpu.make_async_copy(
                w_hbm.at[pl.ds(0, _CHUNK)], rows_v.at[b], gsem.at[b]
            ).wait()

        def issue_out(g, b):
            pltpu.async_copy(
                comp_v.at[b], out_hbm.at[pl.ds(out_base + g * _CHUNK, _CHUNK)],
                osem.at[b],
            )

        def wait_out(b):
            pltpu.make_async_copy(
                comp_v.at[b], out_hbm.at[pl.ds(0, _CHUNK)], osem.at[b]
            ).wait()

        for g0 in range(_AHEAD):
            issue_gather(jnp.int32(g0), g0)

        def outer(p, carry):
            for b in range(_NBUF):
                g = p * _NBUF + b

                @pl.when(g < n_chunks)
                def _():
                    wait_gather(b)

                    g2 = g + _AHEAD
                    b2 = (b + _AHEAD) % _NBUF

                    @pl.when(g2 < n_chunks)
                    def _():
                        @pl.when(g2 >= _NBUF)
                        def _():
                            wait_out(b2)

                        issue_gather(g2, b2)

                    @plsc.parallel_loop(0, _CHUNK, step=1, unroll=8)
                    def _(r):
                        for c in range(D // 16):
                            s = pl.ds(c * 16, 16)
                            comp_v[b, r, s] = rows_v[b, r, s] * scale

                    issue_out(g, b)
            return carry

        lax.fori_loop(0, n_outer, outer, 0)

        for b in range(_NBUF):
            wait_out(b)

    return emb_kernel


def kernel(x, W):
    b, h = x.shape
    V, D = W.shape
    flat = x.reshape(b * h // _IDX_MINOR, _IDX_MINOR).astype(jnp.int32)
    w_pad = jnp.concatenate(
        [W, jnp.zeros((V, 128 - D), dtype=W.dtype)], axis=1
    )
    out = _make(b * h, V, 128, D)(flat, w_pad)
    return out.reshape(b, h, D)


# speedup vs baseline: 1.0567x; 1.0033x over previous
import functools
import math

import jax
import jax.numpy as jnp
from jax import lax
from jax.experimental import pallas as pl
from jax.experimental.pallas import tpu as pltpu
from jax.experimental.pallas import tpu_sc as plsc

_IDX_MINOR = 128
_CHUNK = 128
_NBUF = 4
_NCOMP = 2
_AHEAD = 3


@functools.cache
def _make(B, V, DP, D):
    info = plsc.get_sparse_core_info()
    nw = info.num_cores * info.num_subcores
    b_per_w = B // nw
    idx_rows_per_w = b_per_w // _IDX_MINOR
    n_chunks = b_per_w // _CHUNK
    assert _CHUNK * nw * n_chunks == B
    n_outer = (n_chunks + _NBUF - 1) // _NBUF

    mesh = plsc.VectorSubcoreMesh(core_axis_name="c", subcore_axis_name="s")
    scale = float(math.sqrt(D))

    @functools.partial(
        pl.kernel,
        out_type=jax.ShapeDtypeStruct((B, D), jnp.float32),
        mesh=mesh,
        scratch_types=[
            pltpu.VMEM((idx_rows_per_w, _IDX_MINOR), jnp.int32),
            pltpu.VMEM((_NBUF, _CHUNK, DP), jnp.float32),
            pltpu.VMEM((_NCOMP, _CHUNK, D), jnp.float32),
            pltpu.SemaphoreType.DMA((_NBUF,)),
            pltpu.SemaphoreType.DMA((_NCOMP,)),
        ],
        compiler_params=pltpu.CompilerParams(use_tc_tiling_on_sc=True),
    )
    def emb_kernel(x_hbm, w_hbm, out_hbm, idx_v, rows_v, comp_v, gsem, osem):
        wid = lax.axis_index("s") * info.num_cores + lax.axis_index("c")
        idx_row_base = wid * idx_rows_per_w
        out_base = wid * b_per_w

        pltpu.sync_copy(x_hbm.at[pl.ds(idx_row_base, idx_rows_per_w)], idx_v)

        def issue_gather(g, b):
            pltpu.async_copy(w_hbm.at[idx_v.at[g]], rows_v.at[b], gsem.at[b])

        def wait_gather(b):
            pltpu.make_async_copy(
                w_hbm.at[pl.ds(0, _CHUNK)], rows_v.at[b], gsem.at[b]
            ).wait()

        def issue_out(g, q):
            pltpu.async_copy(
                comp_v.at[q], out_hbm.at[pl.ds(out_base + g * _CHUNK, _CHUNK)],
                osem.at[q],
            )

        def wait_out(q):
            pltpu.make_async_copy(
                comp_v.at[q], out_hbm.at[pl.ds(0, _CHUNK)], osem.at[q]
            ).wait()

        for g0 in range(_AHEAD):
            issue_gather(jnp.int32(g0), g0)

        def outer(p, carry):
            for b in range(_NBUF):
                g = p * _NBUF + b
                q = b % _NCOMP

                @pl.when(g < n_chunks)
                def _():
                    wait_gather(b)

                    g2 = g + _AHEAD
                    b2 = (b + _AHEAD) % _NBUF

                    @pl.when(g2 < n_chunks)
                    def _():
                        issue_gather(g2, b2)

                    @pl.when(g >= _NCOMP)
                    def _():
                        wait_out(q)

                    @plsc.parallel_loop(0, _CHUNK, step=1, unroll=8)
                    def _(r):
                        for c in range(D // 16):
                            s = pl.ds(c * 16, 16)
                            comp_v[q, r, s] = rows_v[b, r, s] * scale

                    issue_out(g, q)
            return carry

        lax.fori_loop(0, n_outer, outer, 0)

        for q in range(_NCOMP):
            wait_out(q)

    return emb_kernel


def kernel(x, W):
    b, h = x.shape
    V, D = W.shape
    flat = x.reshape(b * h // _IDX_MINOR, _IDX_MINOR).astype(jnp.int32)
    w_pad = jnp.concatenate(
        [W, jnp.zeros((V, 128 - D), dtype=W.dtype)], axis=1
    )
    out = _make(b * h, V, 128, D)(flat, w_pad)
    return out.reshape(b, h, D)
